# baseline (speedup 1.0000x reference)
import jax
import jax.numpy as jnp
from jax.experimental import pallas as pl
from jax.experimental.pallas import tpu as pltpu

_BLOCK = 2048
_RFF_CHUNK = 512


def _bdot(a, b):
    return jax.lax.dot(a, b, preferred_element_type=jnp.float32)


def _fused_mlp_kernel(x_ref, kt_ref, a0_ref, b0_ref, a1_ref, b1_ref,
                      a2_ref, b2_ref, a3_ref, b3_ref, a4_ref, b4_ref, y_ref,
                      fa_ref, fb_ref):
    f_a = fa_ref[...]
    f_b = fb_ref[...]

    x = x_ref[...]
    kt = kt_ref[...]

    def rff_chunk(j):
        lo = j * _RFF_CHUNK
        xc = x[lo:lo + _RFF_CHUNK, :]
        z = (xc[:, 0:1] * kt[0:1, :]
             + xc[:, 1:2] * kt[1:2, :]
             + xc[:, 2:3] * kt[2:3, :])
        dst = fa_ref if lo < _BLOCK else fb_ref
        dlo = lo if lo < _BLOCK else lo - _BLOCK
        dst[dlo:dlo + _RFF_CHUNK, 0:128] = jnp.cos(z).astype(jnp.bfloat16)
        dst[dlo:dlo + _RFF_CHUNK, 128:256] = jnp.sin(z).astype(jnp.bfloat16)

    nc = (2 * _BLOCK) // _RFF_CHUNK

    a0 = a0_ref[...]
    h_a = jnp.tanh(_bdot(f_a, a0) + b0_ref[...])
    rff_chunk(0)
    h_b = jnp.tanh(_bdot(f_b, a0) + b0_ref[...])
    rff_chunk(1)
    a1 = a1_ref[...]
    h_a = jnp.tanh(_bdot(h_a.astype(jnp.bfloat16), a1) + b1_ref[...])
    rff_chunk(2)
    h_b = jnp.tanh(_bdot(h_b.astype(jnp.bfloat16), a1) + b1_ref[...])
    rff_chunk(3)
    a2 = a2_ref[...]
    h_a = jnp.tanh(_bdot(h_a.astype(jnp.bfloat16), a2) + b2_ref[...])
    rff_chunk(4)
    h_b = jnp.tanh(_bdot(h_b.astype(jnp.bfloat16), a2) + b2_ref[...])
    rff_chunk(5)
    a3 = a3_ref[...]
    h_a = jnp.tanh(_bdot(h_a.astype(jnp.bfloat16), a3) + b3_ref[...])
    rff_chunk(6)
    h_b = jnp.tanh(_bdot(h_b.astype(jnp.bfloat16), a3) + b3_ref[...])
    rff_chunk(7)
    a4 = a4_ref[...]
    y_ref[0:_BLOCK, :] = _bdot(h_a.astype(jnp.bfloat16), a4) + b4_ref[...]
    y_ref[_BLOCK:2 * _BLOCK, :] = (_bdot(h_b.astype(jnp.bfloat16), a4)
                                   + b4_ref[...])


@jax.jit
def kernel(x, kernel_rff, W0, b0, W1, b1, W2, b2, W3, b3, W4, b4):
    n = x.shape[0]
    step_rows = 2 * _BLOCK
    nb = n // step_rows
    kt = kernel_rff.T
    bf = jnp.bfloat16
    a0 = W0.T.astype(bf)
    a1, a2, a3, a4 = (W1.T.astype(bf), W2.T.astype(bf), W3.T.astype(bf),
                      W4.T.astype(bf))
    grid = (nb + 1,)

    def rows_in(i):
        return (jnp.minimum(i, nb - 1), 0)

    def rows_out(i):
        return (jnp.maximum(i - 1, 0), 0)

    def whole(i):
        return (0, 0)

    full = lambda arr: pl.BlockSpec(arr.shape, whole)
    out = pl.pallas_call(
        _fused_mlp_kernel,
        grid=grid,
        in_specs=[
            pl.BlockSpec((step_rows, 3), rows_in),
            full(kt),
            full(a0), pl.BlockSpec((1, b0.shape[0]), whole),
            full(a1), pl.BlockSpec((1, b1.shape[0]), whole),
            full(a2), pl.BlockSpec((1, b2.shape[0]), whole),
            full(a3), pl.BlockSpec((1, b3.shape[0]), whole),
            full(a4), pl.BlockSpec((1, b4.shape[0]), whole),
        ],
        out_specs=pl.BlockSpec((step_rows, 3), rows_out),
        out_shape=jax.ShapeDtypeStruct((n, 3), x.dtype),
        scratch_shapes=[pltpu.VMEM((_BLOCK, 256), jnp.bfloat16),
                        pltpu.VMEM((_BLOCK, 256), jnp.bfloat16)],
        compiler_params=pltpu.CompilerParams(
            dimension_semantics=("arbitrary",),
        ),
    )(x, kt, a0, b0[None, :], a1, b1[None, :], a2, b2[None, :],
      a3, b3[None, :], a4, b4[None, :])
    return out


# speedup vs baseline: 1.0629x; 1.0629x over previous
import jax
import jax.numpy as jnp
from jax.experimental import pallas as pl
from jax.experimental.pallas import tpu as pltpu

_BLOCK = 2048
_RFF_CHUNK = 512


def _bdot(a, b):
    return jax.lax.dot(a, b, preferred_element_type=jnp.float32)


def _fused_mlp_kernel(x_ref, kt_ref, a0_ref, b0_ref, a1_ref, b1_ref,
                      a2_ref, b2_ref, a3_ref, b3_ref, a4_ref, b4_ref, y_ref,
                      f_ref):
    f = f_ref[...]

    x = x_ref[...]
    kt = kt_ref[...]

    def rff_chunk(j):
        lo = j * _RFF_CHUNK
        xc = x[lo:lo + _RFF_CHUNK, :]
        z = (xc[:, 0:1] * kt[0:1, :]
             + xc[:, 1:2] * kt[1:2, :]
             + xc[:, 2:3] * kt[2:3, :])
        f_ref[lo:lo + _RFF_CHUNK, 0:128] = jnp.cos(z).astype(jnp.bfloat16)
        f_ref[lo:lo + _RFF_CHUNK, 128:256] = jnp.sin(z).astype(jnp.bfloat16)

    h = jnp.tanh(_bdot(f, a0_ref[...]) + b0_ref[...])
    rff_chunk(0)
    h = jnp.tanh(_bdot(h.astype(jnp.bfloat16), a1_ref[...]) + b1_ref[...])
    rff_chunk(1)
    h = jnp.tanh(_bdot(h.astype(jnp.bfloat16), a2_ref[...]) + b2_ref[...])
    rff_chunk(2)
    h = jnp.tanh(_bdot(h.astype(jnp.bfloat16), a3_ref[...]) + b3_ref[...])
    rff_chunk(3)
    y_ref[...] = _bdot(h.astype(jnp.bfloat16), a4_ref[...]) + b4_ref[...]


@jax.jit
def kernel(x, kernel_rff, W0, b0, W1, b1, W2, b2, W3, b3, W4, b4):
    n = x.shape[0]
    nb = n // _BLOCK
    kt = kernel_rff.T
    bf = jnp.bfloat16
    a0 = W0.T.astype(bf)
    a1, a2, a3, a4 = (W1.T.astype(bf), W2.T.astype(bf), W3.T.astype(bf),
                      W4.T.astype(bf))
    grid = (nb + 1,)

    def rows_in(i):
        return (jnp.minimum(i, nb - 1), 0)

    def rows_out(i):
        return (jnp.maximum(i - 1, 0), 0)

    def whole(i):
        return (0, 0)

    full = lambda arr: pl.BlockSpec(arr.shape, whole)
    out = pl.pallas_call(
        _fused_mlp_kernel,
        grid=grid,
        in_specs=[
            pl.BlockSpec((_BLOCK, 3), rows_in),
            full(kt),
            full(a0), pl.BlockSpec((1, b0.shape[0]), whole),
            full(a1), pl.BlockSpec((1, b1.shape[0]), whole),
            full(a2), pl.BlockSpec((1, b2.shape[0]), whole),
            full(a3), pl.BlockSpec((1, b3.shape[0]), whole),
            full(a4), pl.BlockSpec((1, b4.shape[0]), whole),
        ],
        out_specs=pl.BlockSpec((_BLOCK, 3), rows_out),
        out_shape=jax.ShapeDtypeStruct((n, 3), x.dtype),
        scratch_shapes=[pltpu.VMEM((_BLOCK, 256), jnp.bfloat16)],
        compiler_params=pltpu.CompilerParams(
            dimension_semantics=("arbitrary",),
        ),
    )(x, kt, a0, b0[None, :], a1, b1[None, :], a2, b2[None, :],
      a3, b3[None, :], a4, b4[None, :])
    return out
